# Initial kernel scaffold; baseline (speedup 1.0000x reference)
import jax
import jax.numpy as jnp
from jax.experimental import pallas as pl


def kernel(x, edge_index, W0, a_src0, a_dst0, b0, g0, be0, W1, a_src1, a_dst1, b1, g1, be1, W2, a_src2, a_dst2, b2):
    raise NotImplementedError("write your pallas kernel here")


# speedup vs baseline: 2.5196x; 2.5196x over previous
import functools

import jax
import jax.numpy as jnp
from jax.experimental import pallas as pl

N = 10000
E = 160000
D_IN = 256
HID = 128
HEADS = 4
D_OUT = 128
EPS = 1e-5

_ROW_BLK = 400


def _dense_body(x_ref, w_ref, scale_ref, shift_ref, h_ref, *, apply_act):
    y = x_ref[...]
    if apply_act:
        y = jnp.maximum(y * scale_ref[...] + shift_ref[...], 0.0)
    h_ref[...] = jnp.dot(y, w_ref[...], preferred_element_type=jnp.float32)


def _dense_layer(x, w, a_src, a_dst, scale, shift, apply_act):
    n, k = x.shape
    m = w.shape[1]
    heads = a_src.shape[0]
    grid = (n // _ROW_BLK,)
    h = pl.pallas_call(
        functools.partial(_dense_body, apply_act=apply_act),
        grid=grid,
        in_specs=[
            pl.BlockSpec((_ROW_BLK, k), lambda i: (i, 0)),
            pl.BlockSpec((k, m), lambda i: (0, 0)),
            pl.BlockSpec((1, k), lambda i: (0, 0)),
            pl.BlockSpec((1, k), lambda i: (0, 0)),
        ],
        out_specs=pl.BlockSpec((_ROW_BLK, m), lambda i: (i, 0)),
        out_shape=jax.ShapeDtypeStruct((n, m), jnp.float32),
    )(x, w, scale, shift)
    hr = h.reshape(n, heads, m // heads)
    s = (hr * a_src[None, :, :]).sum(-1)
    d = (hr * a_dst[None, :, :]).sum(-1)
    return h, s, d


def _edge_softmax_agg(h, alpha_s, alpha_d, src, dst, heads, hid):
    n = h.shape[0]
    e = alpha_s[src] + alpha_d[dst]
    e = jnp.where(e > 0, e, 0.2 * e)
    e_max = jax.ops.segment_max(e, dst, num_segments=n)
    e_max = jnp.where(jnp.isfinite(e_max), e_max, 0.0)
    ex = jnp.exp(e - e_max[dst])
    denom = jax.ops.segment_sum(ex, dst, num_segments=n)
    alpha = ex / (denom[dst] + 1e-16)
    alpha_w = jnp.repeat(alpha, hid, axis=1)
    msg = h[src] * alpha_w
    return jax.ops.segment_sum(msg, dst, num_segments=n)


def kernel(x, edge_index, W0, a_src0, a_dst0, b0, g0, be0,
           W1, a_src1, a_dst1, b1, g1, be1, W2, a_src2, a_dst2, b2):
    src = edge_index[0]
    dst = edge_index[1]
    one_k = jnp.ones((1, x.shape[1]), jnp.float32)
    zero_k = jnp.zeros((1, x.shape[1]), jnp.float32)

    g0p = g0 / jnp.sqrt(1.0 + EPS)
    scale1 = g0p[None, :]
    shift1 = (b0 * g0p + be0)[None, :]
    g1p = g1 / jnp.sqrt(1.0 + EPS)
    scale2 = g1p[None, :]
    shift2 = (b1 * g1p + be1)[None, :]

    h0, s0, d0 = _dense_layer(x, W0, a_src0, a_dst0, one_k, zero_k, False)
    agg0 = _edge_softmax_agg(h0, s0, d0, src, dst, HEADS, HID)

    h1, s1, d1 = _dense_layer(agg0, W1, a_src1, a_dst1, scale1, shift1, True)
    agg1 = _edge_softmax_agg(h1, s1, d1, src, dst, HEADS, HID)

    h2, s2, d2 = _dense_layer(agg1, W2, a_src2, a_dst2, scale2, shift2, True)
    agg2 = _edge_softmax_agg(h2, s2, d2, src, dst, 1, D_OUT)

    return agg2 + b2


# speedup vs baseline: 16.5977x; 6.5876x over previous
import functools

import jax
import jax.numpy as jnp
from jax import lax
from jax.experimental import pallas as pl
from jax.experimental.pallas import tpu as pltpu
from jax.experimental.pallas import tpu_sc as plsc

N = 10000
E = 160000
D_IN = 256
HID = 128
HEADS = 4
D_OUT = 128
EPS = 1e-5

_ROW_BLK = 400
_NT = 16
_NPAD = 10240
_NSL = _NPAD // _NT
_WIN = 128
_NWIN = E // _WIN


def _dense_body(x_ref, w_ref, as_ref, ad_ref, scale_ref, shift_ref,
                h_ref, s_ref, d_ref, smax_ref, *, apply_act):
    i = pl.program_id(0)
    y = x_ref[...]
    if apply_act:
        y = jnp.maximum(y * scale_ref[...] + shift_ref[...], 0.0)
    h = jnp.dot(y, w_ref[...], preferred_element_type=jnp.float32)
    h_ref[...] = h
    s = jnp.dot(h, as_ref[...], preferred_element_type=jnp.float32)
    d = jnp.dot(h, ad_ref[...], preferred_element_type=jnp.float32)
    s_ref[...] = s
    d_ref[...] = d
    m = jnp.max(s, axis=0, keepdims=True)
    prev = jnp.where(i == 0, jnp.full_like(m, -jnp.inf), smax_ref[...])
    smax_ref[...] = jnp.maximum(prev, m)


def _dense_layer(x, w, a_src, a_dst, scale, shift, apply_act):
    n, k = x.shape
    m = w.shape[1]
    heads = a_src.shape[0]
    eye = jnp.eye(heads, dtype=jnp.float32)
    as_mat = (a_src[:, :, None] * eye[:, None, :]).reshape(m, heads)
    ad_mat = (a_dst[:, :, None] * eye[:, None, :]).reshape(m, heads)
    if apply_act:
        sc_spec = pl.BlockSpec((_ROW_BLK, k), lambda i: (i, 0))
    else:
        sc_spec = pl.BlockSpec((1, k), lambda i: (0, 0))
    return pl.pallas_call(
        functools.partial(_dense_body, apply_act=apply_act),
        grid=(n // _ROW_BLK,),
        in_specs=[
            pl.BlockSpec((_ROW_BLK, k), lambda i: (i, 0)),
            pl.BlockSpec((k, m), lambda i: (0, 0)),
            pl.BlockSpec((m, heads), lambda i: (0, 0)),
            pl.BlockSpec((m, heads), lambda i: (0, 0)),
            sc_spec,
            pl.BlockSpec((1, k), lambda i: (0, 0)),
        ],
        out_specs=[
            pl.BlockSpec((_ROW_BLK, m), lambda i: (i, 0)),
            pl.BlockSpec((_ROW_BLK, heads), lambda i: (i, 0)),
            pl.BlockSpec((_ROW_BLK, heads), lambda i: (i, 0)),
            pl.BlockSpec((1, heads), lambda i: (0, 0)),
        ],
        out_shape=[
            jax.ShapeDtypeStruct((n, m), jnp.float32),
            jax.ShapeDtypeStruct((n, heads), jnp.float32),
            jax.ShapeDtypeStruct((n, heads), jnp.float32),
            jax.ShapeDtypeStruct((1, heads), jnp.float32),
        ],
    )(x, w, as_mat, ad_mat, scale, shift)


def _sc_agg_body(h4, ei_flat, st_flat, dt_flat, smax_flat, z2d, z1d,
                 raw_out, den_out,
                 s_tbl, d_tbl, sm_buf, src_w, dst_w, rows, wvec,
                 acc_sh, den_sh, sem,
                 *, cpc, split_edges):
    c = lax.axis_index("c")
    t = lax.axis_index("s")
    for j in range(cpc):
        head = c * cpc + j if not split_edges else 0
        orow = head if not split_edges else c
        pltpu.sync_copy(st_flat.at[pl.ds(head * _NPAD, _NPAD)], s_tbl)
        pltpu.sync_copy(dt_flat.at[pl.ds(head * _NPAD, _NPAD)], d_tbl)
        pltpu.sync_copy(smax_flat.at[pl.ds(head * 128, 16)], sm_buf)
        s16 = sm_buf[...]
        pltpu.sync_copy(z2d.at[pl.ds(t * _NSL, _NSL)],
                        acc_sh.at[pl.ds(t * _NSL, _NSL)])

        @pl.when(t == 0)
        def _():
            pltpu.sync_copy(z1d, den_sh)

        plsc.subcore_barrier()

        if split_edges:
            wid = c * _NT + t
            stride = 32
        else:
            wid = t
            stride = 16
        trips = (_NWIN + stride - 1) // stride

        def window(wi, _):
            widx = wi * stride + wid

            @pl.when(widx < _NWIN)
            def _():
                off = widx * _WIN
                pltpu.sync_copy(ei_flat.at[pl.ds(off, _WIN)], src_w)
                pltpu.sync_copy(ei_flat.at[pl.ds(E + off, _WIN)], dst_w)
                pltpu.async_copy(h4.at[head].at[src_w], rows, sem).wait()

                def group(g, _):
                    si = src_w[pl.ds(g * 16, 16)]
                    di = dst_w[pl.ds(g * 16, 16)]
                    sg = plsc.load_gather(s_tbl, [si])
                    dg = plsc.load_gather(d_tbl, [di])
                    e16 = sg + dg
                    e16 = jnp.where(e16 > 0, e16, 0.2 * e16)
                    m16 = jnp.maximum(s16 + dg, 0.0)
                    w16 = jnp.exp(e16 - m16)
                    wvec[pl.ds(g * 16, 16)] = w16
                    for r in range(16):
                        bidx = jnp.zeros((16,), jnp.int32) + (g * 16 + r)
                        wb = plsc.load_gather(wvec, [bidx])
                        for f in range(HID // 16):
                            rows[g * 16 + r, pl.ds(f * 16, 16)] = (
                                rows[g * 16 + r, pl.ds(f * 16, 16)] * wb)
                    return 0

                lax.fori_loop(0, _WIN // 16, group, 0)
                pltpu.sync_copy(wvec, den_sh.at[dst_w], add=True)
                pltpu.sync_copy(rows, acc_sh.at[dst_w], add=True)

            return 0

        lax.fori_loop(0, trips, window, 0)
        plsc.subcore_barrier()
        pltpu.sync_copy(acc_sh.at[pl.ds(t * _NSL, _NSL)],
                        raw_out.at[orow].at[pl.ds(t * _NSL, _NSL)])

        @pl.when(t == 0)
        def _():
            pltpu.sync_copy(den_sh, den_out.at[pl.ds(orow * _NPAD, _NPAD)])

        plsc.subcore_barrier()


def _sc_agg(h4, ei_flat, st_flat, dt_flat, smax_flat, heads, split_edges):
    if split_edges:
        cpc, rrows = 1, 2
    else:
        cpc, rrows = heads // 2, heads
    z2d = jnp.zeros((_NPAD, HID), jnp.float32)
    z1d = jnp.zeros((_NPAD,), jnp.float32)
    mesh = plsc.VectorSubcoreMesh(core_axis_name="c", subcore_axis_name="s")
    return pl.kernel(
        functools.partial(_sc_agg_body, cpc=cpc, split_edges=split_edges),
        mesh=mesh,
        compiler_params=pltpu.CompilerParams(needs_layout_passes=False),
        out_type=[
            jax.ShapeDtypeStruct((rrows, _NPAD, HID), jnp.float32),
            jax.ShapeDtypeStruct((rrows * _NPAD,), jnp.float32),
        ],
        scratch_types=[
            pltpu.VMEM((_NPAD,), jnp.float32),
            pltpu.VMEM((_NPAD,), jnp.float32),
            pltpu.VMEM((16,), jnp.float32),
            pltpu.VMEM((_WIN,), jnp.int32),
            pltpu.VMEM((_WIN,), jnp.int32),
            pltpu.VMEM((_WIN, HID), jnp.float32),
            pltpu.VMEM((_WIN,), jnp.float32),
            pltpu.VMEM_SHARED((_NPAD, HID), jnp.float32),
            pltpu.VMEM_SHARED((_NPAD,), jnp.float32),
            pltpu.SemaphoreType.DMA,
        ],
    )(h4, ei_flat, st_flat, dt_flat, smax_flat, z2d, z1d)


def _chunked(h, heads):
    return jnp.transpose(h.reshape(N, heads, HID), (1, 0, 2))


def _sc_inputs(s, d, smax):
    pad = ((0, 0), (0, _NPAD - N))
    st = jnp.pad(jnp.transpose(s), pad).ravel()
    dt = jnp.pad(jnp.transpose(d), pad).ravel()
    sm = jnp.repeat(jnp.transpose(smax), 128, axis=1).ravel()
    return st, dt, sm


def kernel(x, edge_index, W0, a_src0, a_dst0, b0, g0, be0,
           W1, a_src1, a_dst1, b1, g1, be1, W2, a_src2, a_dst2, b2):
    ei_flat = edge_index.ravel()
    one_k = jnp.ones((1, x.shape[1]), jnp.float32)
    zero_k = jnp.zeros((1, x.shape[1]), jnp.float32)
    g0p = g0 / jnp.sqrt(1.0 + EPS)
    g1p = g1 / jnp.sqrt(1.0 + EPS)

    h0, s0, d0, smax0 = _dense_layer(x, W0, a_src0, a_dst0, one_k, zero_k, False)
    st0, dt0, sm0 = _sc_inputs(s0, d0, smax0)
    raw0, den0 = _sc_agg(_chunked(h0, HEADS), ei_flat, st0, dt0, sm0, HEADS, False)
    raw0_2d = jnp.transpose(raw0[:, :N], (1, 0, 2)).reshape(N, HEADS * HID)
    den0 = den0.reshape(HEADS, _NPAD)[:, :N]
    invd0 = jnp.repeat(jnp.transpose(1.0 / (den0 + 1e-16)), HID, axis=1)
    scale1 = invd0 * g0p[None, :]
    shift1 = (b0 * g0p + be0)[None, :]

    h1, s1, d1, smax1 = _dense_layer(raw0_2d, W1, a_src1, a_dst1,
                                     scale1, shift1, True)
    st1, dt1, sm1 = _sc_inputs(s1, d1, smax1)
    raw1, den1 = _sc_agg(_chunked(h1, HEADS), ei_flat, st1, dt1, sm1, HEADS, False)
    raw1_2d = jnp.transpose(raw1[:, :N], (1, 0, 2)).reshape(N, HEADS * HID)
    den1 = den1.reshape(HEADS, _NPAD)[:, :N]
    invd1 = jnp.repeat(jnp.transpose(1.0 / (den1 + 1e-16)), HID, axis=1)
    scale2 = invd1 * g1p[None, :]
    shift2 = (b1 * g1p + be1)[None, :]

    h2, s2, d2, smax2 = _dense_layer(raw1_2d, W2, a_src2, a_dst2,
                                     scale2, shift2, True)
    st2, dt2, sm2 = _sc_inputs(s2, d2, smax2)
    raw2, den2 = _sc_agg(h2[None, :, :], ei_flat, st2, dt2, sm2, 1, True)
    den2 = den2.reshape(2, _NPAD)
    den = den2[0, :N] + den2[1, :N]
    out = (raw2[0, :N] + raw2[1, :N]) / (den + 1e-16)[:, None]
    return out + b2
